# baseline (speedup 1.0000x reference)
import functools

import jax
import jax.numpy as jnp
from jax import lax
from jax.experimental import pallas as pl
from jax.experimental.pallas import tpu as pltpu
from jax.experimental.pallas import tpu_sc as plsc

B = 16384
D = 64
N_AGE = 100
N_INCOME = 20
NC = 2
NS = 16
NW = NC * NS
BPW = B // NW
NCHUNK = BPW // 128

A_PAD = 112
I_PAD = 32

_mesh = plsc.VectorSubcoreMesh(core_axis_name="c", subcore_axis_name="s")


@functools.partial(
    pl.kernel,
    mesh=_mesh,
    out_type=jax.ShapeDtypeStruct((B,), jnp.float32),
    compiler_params=pltpu.CompilerParams(
        needs_layout_passes=False, use_tc_tiling_on_sc=False),
    scratch_types=[
        pltpu.VMEM((NCHUNK, 128), jnp.int32),
        pltpu.VMEM((NCHUNK, 128), jnp.int32),
        pltpu.VMEM((BPW,), jnp.int32),
        pltpu.VMEM((BPW,), jnp.int32),
        pltpu.VMEM((264,), jnp.float32),
        pltpu.VMEM((BPW, D), jnp.float32),
        pltpu.VMEM((BPW, D), jnp.float32),
        pltpu.VMEM((A_PAD, D), jnp.float32),
        pltpu.VMEM((I_PAD, D), jnp.float32),
        pltpu.VMEM((A_PAD,), jnp.float32),
        pltpu.VMEM((I_PAD,), jnp.float32),
        pltpu.VMEM((BPW, 16), jnp.float32),
        pltpu.VMEM((BPW,), jnp.float32),
        pltpu.SemaphoreType.DMA,
    ],
)
def _sc_recommender(user_hbm, item_hbm, age_hbm, inc_hbm,
                    utab_hbm, itab_hbm, atab_hbm, ntab_hbm, w_hbm,
                    out_hbm,
                    uidx_v, iidx_v, aidx_v, nidx_v, w_v,
                    urows_v, irows_v, atab_v, ntab_v,
                    proja_v, projn_v, part_v, out_v, sem):
    wid = lax.axis_index("s") * NC + lax.axis_index("c")
    base = wid * BPW
    ri = lax.iota(jnp.int32, 16)

    for c in range(NCHUNK):
        pltpu.sync_copy(user_hbm.at[pl.ds(base + c * 128, 128)], uidx_v.at[c])
        pltpu.sync_copy(item_hbm.at[pl.ds(base + c * 128, 128)], iidx_v.at[c])
    pltpu.sync_copy(age_hbm.at[pl.ds(base, BPW)], aidx_v)
    pltpu.sync_copy(inc_hbm.at[pl.ds(base, BPW)], nidx_v)
    pltpu.sync_copy(w_hbm, w_v)
    pltpu.sync_copy(atab_hbm, atab_v.at[pl.ds(0, N_AGE)])
    pltpu.sync_copy(ntab_hbm, ntab_v.at[pl.ds(0, N_INCOME)])

    copies = []
    for c in range(NCHUNK):
        copies.append(pltpu.async_copy(
            utab_hbm.at[uidx_v.at[c]], urows_v.at[pl.ds(c * 128, 128)], sem))
        copies.append(pltpu.async_copy(
            itab_hbm.at[iidx_v.at[c]], irows_v.at[pl.ds(c * 128, 128)], sem))

    NGA = A_PAD // 16
    NGI = I_PAD // 16
    zero = jnp.zeros((16,), jnp.float32)

    def proj_body(d, accs):
        fd = jnp.full((16,), d, jnp.int32)
        wa = plsc.load_gather(w_v, [jnp.full((16,), 128 + d, jnp.int32)])
        wn = plsc.load_gather(w_v, [jnp.full((16,), 192 + d, jnp.int32)])
        out = []
        for g in range(NGA):
            col = plsc.load_gather(atab_v, [g * 16 + ri, fd])
            out.append(accs[g] + col * wa)
        for g in range(NGI):
            col = plsc.load_gather(ntab_v, [g * 16 + ri, fd])
            out.append(accs[NGA + g] + col * wn)
        return tuple(out)

    accs = lax.fori_loop(0, D, proj_body, (zero,) * (NGA + NGI))
    for g in range(NGA):
        proja_v[pl.ds(g * 16, 16)] = accs[g]
    for g in range(NGI):
        projn_v[pl.ds(g * 16, 16)] = accs[NGA + g]

    for cp in copies:
        cp.wait()

    wu = [w_v[pl.ds(16 * k, 16)] for k in range(4)]
    wi = [w_v[pl.ds(64 + 16 * k, 16)] for k in range(4)]

    def p1_body(r, carry):
        p = urows_v[r, pl.ds(0, 16)] * wu[0]
        for k in range(1, 4):
            p = p + urows_v[r, pl.ds(16 * k, 16)] * wu[k]
        for k in range(4):
            p = p + irows_v[r, pl.ds(16 * k, 16)] * wi[k]
        part_v[r, :] = p
        return carry

    lax.fori_loop(0, BPW, p1_body, 0)

    bias = plsc.load_gather(w_v, [jnp.full((16,), 256, jnp.int32)])

    def p2_body(g, carry):
        rb = g * 16
        ridx = rb + ri
        acc = bias
        for l in range(16):
            acc = acc + plsc.load_gather(part_v, [ridx, jnp.full((16,), l, jnp.int32)])
        a_i = aidx_v[pl.ds(rb, 16)]
        n_i = nidx_v[pl.ds(rb, 16)]
        acc = acc + plsc.load_gather(proja_v, [a_i])
        acc = acc + plsc.load_gather(projn_v, [n_i])
        out_v[pl.ds(rb, 16)] = 1.0 / (1.0 + jnp.exp(-acc))
        return carry

    lax.fori_loop(0, BPW // 16, p2_body, 0)

    pltpu.sync_copy(out_v, out_hbm.at[pl.ds(base, BPW)])


def kernel(user, item, age, income, user_table, item_table,
           age_table, income_table, fc_w, fc_b):
    w = jnp.concatenate([
        fc_w.reshape(-1).astype(jnp.float32),
        fc_b.reshape(-1).astype(jnp.float32),
        jnp.zeros((7,), jnp.float32),
    ])
    return _sc_recommender(
        user.astype(jnp.int32), item.astype(jnp.int32),
        age.astype(jnp.int32), income.astype(jnp.int32),
        user_table, item_table, age_table, income_table, w)


# speedup vs baseline: 1.6200x; 1.6200x over previous
import functools

import jax
import jax.numpy as jnp
from jax import lax
from jax.experimental import pallas as pl
from jax.experimental.pallas import tpu as pltpu
from jax.experimental.pallas import tpu_sc as plsc

B = 16384
D = 64
N_AGE = 100
N_INCOME = 20
NC = 2
NS = 16
NW = NC * NS
BPW = B // NW
QR = BPW // 4
QB = QR // 16

A_PAD = 112
I_PAD = 32

_mesh = plsc.VectorSubcoreMesh(core_axis_name="c", subcore_axis_name="s")


@functools.partial(
    pl.kernel,
    mesh=_mesh,
    out_type=jax.ShapeDtypeStruct((B,), jnp.float32),
    compiler_params=pltpu.CompilerParams(needs_layout_passes=False),
    scratch_types=[
        pltpu.VMEM((BPW,), jnp.int32),
        pltpu.VMEM((BPW,), jnp.int32),
        pltpu.VMEM((BPW,), jnp.int32),
        pltpu.VMEM((BPW,), jnp.int32),
        pltpu.VMEM((264,), jnp.float32),
        pltpu.VMEM((QR, D), jnp.float32),
        pltpu.VMEM((QR, D), jnp.float32),
        pltpu.VMEM((QR, D), jnp.float32),
        pltpu.VMEM((QR, D), jnp.float32),
        pltpu.VMEM((A_PAD, D), jnp.float32),
        pltpu.VMEM((I_PAD, D), jnp.float32),
        pltpu.VMEM((A_PAD,), jnp.float32),
        pltpu.VMEM((I_PAD,), jnp.float32),
        pltpu.VMEM((BPW * 16,), jnp.float32),
        pltpu.VMEM((BPW,), jnp.float32),
        pltpu.SemaphoreType.DMA,
        pltpu.SemaphoreType.DMA,
        pltpu.SemaphoreType.DMA,
    ],
)
def _sc_recommender(user_hbm, item_hbm, age_hbm, inc_hbm,
                    utab_hbm, itab_hbm, atab_hbm, ntab_hbm, w_hbm,
                    out_hbm,
                    uidx_v, iidx_v, aidx_v, nidx_v, w_v,
                    ur_a, ur_b, ir_a, ir_b, atab_v, ntab_v,
                    proja_v, projn_v, part_v, out_v,
                    sem_t, sem_u, sem_i):
    wid = lax.axis_index("s") * NC + lax.axis_index("c")
    base = wid * BPW
    ri = lax.iota(jnp.int32, 16)

    pltpu.async_copy(atab_hbm, atab_v.at[pl.ds(0, N_AGE), :], sem_t)
    pltpu.async_copy(ntab_hbm, ntab_v.at[pl.ds(0, N_INCOME), :], sem_t)
    pltpu.sync_copy(user_hbm.at[pl.ds(base, BPW)], uidx_v)
    pltpu.sync_copy(item_hbm.at[pl.ds(base, BPW)], iidx_v)
    pltpu.sync_copy(age_hbm.at[pl.ds(base, BPW)], aidx_v)
    pltpu.sync_copy(inc_hbm.at[pl.ds(base, BPW)], nidx_v)
    pltpu.sync_copy(w_hbm, w_v)

    def issue_quarter(q, ubuf, ibuf):
        def blk(b, carry):
            gr = q * QR + b * 16
            uv = uidx_v[pl.ds(gr, 16)]
            iv = iidx_v[pl.ds(gr, 16)]
            for j in range(16):
                lr = b * 16 + j
                pltpu.async_copy(utab_hbm.at[pl.ds(uv[j], 1), :],
                                 ubuf.at[pl.ds(lr, 1), :], sem_u)
                pltpu.async_copy(itab_hbm.at[pl.ds(iv[j], 1), :],
                                 ibuf.at[pl.ds(lr, 1), :], sem_i)
            return carry

        lax.fori_loop(0, QB, blk, 0)

    def drain_quarter(ubuf, ibuf):
        def drow(r, carry):
            pltpu.make_async_copy(utab_hbm.at[pl.ds(0, 1), :],
                                  ubuf.at[pl.ds(r, 1), :], sem_u).wait()
            pltpu.make_async_copy(itab_hbm.at[pl.ds(0, 1), :],
                                  ibuf.at[pl.ds(r, 1), :], sem_i).wait()
            return carry

        lax.fori_loop(0, QR, drow, 0)

    wu = [w_v[pl.ds(16 * k, 16)] for k in range(4)]
    wi = [w_v[pl.ds(64 + 16 * k, 16)] for k in range(4)]

    def pass1_quarter(q, ubuf, ibuf):
        def p1(r, carry):
            p = ubuf[r, pl.ds(0, 16)] * wu[0]
            for k in range(1, 4):
                p = p + ubuf[r, pl.ds(16 * k, 16)] * wu[k]
            for k in range(4):
                p = p + ibuf[r, pl.ds(16 * k, 16)] * wi[k]
            part_v[pl.ds((q * QR + r) * 16, 16)] = p
            return carry

        lax.fori_loop(0, QR, p1, 0)

    issue_quarter(0, ur_a, ir_a)
    issue_quarter(1, ur_b, ir_b)

    pltpu.make_async_copy(atab_hbm, atab_v.at[pl.ds(0, N_AGE), :],
                          sem_t).wait()
    pltpu.make_async_copy(ntab_hbm, ntab_v.at[pl.ds(0, N_INCOME), :],
                          sem_t).wait()

    NGA = A_PAD // 16
    NGI = I_PAD // 16
    zero = jnp.zeros((16,), jnp.float32)

    def proj_body(d, accs):
        fd = jnp.full((16,), d, jnp.int32)
        wa = plsc.load_gather(w_v, [jnp.full((16,), 128 + d, jnp.int32)])
        wn = plsc.load_gather(w_v, [jnp.full((16,), 192 + d, jnp.int32)])
        out = []
        for g in range(NGA):
            col = plsc.load_gather(atab_v, [g * 16 + ri, fd])
            out.append(accs[g] + col * wa)
        for g in range(NGI):
            col = plsc.load_gather(ntab_v, [g * 16 + ri, fd])
            out.append(accs[NGA + g] + col * wn)
        return tuple(out)

    accs = lax.fori_loop(0, D, proj_body, (zero,) * (NGA + NGI))
    for g in range(NGA):
        proja_v[pl.ds(g * 16, 16)] = accs[g]
    for g in range(NGI):
        projn_v[pl.ds(g * 16, 16)] = accs[NGA + g]

    bufs = [(ur_a, ir_a), (ur_b, ir_b)]
    for q in range(4):
        ubuf, ibuf = bufs[q % 2]
        drain_quarter(ubuf, ibuf)
        pass1_quarter(q, ubuf, ibuf)
        if q + 2 < 4:
            issue_quarter(q + 2, ubuf, ibuf)

    bias = plsc.load_gather(w_v, [jnp.full((16,), 256, jnp.int32)])

    def p2_body(g, carry):
        rb = g * 16
        pidx = (rb + ri) * 16
        acc = bias
        for l in range(16):
            acc = acc + plsc.load_gather(part_v, [pidx + l])
        a_i = aidx_v[pl.ds(rb, 16)]
        n_i = nidx_v[pl.ds(rb, 16)]
        acc = acc + plsc.load_gather(proja_v, [a_i])
        acc = acc + plsc.load_gather(projn_v, [n_i])
        out_v[pl.ds(rb, 16)] = 1.0 / (1.0 + jnp.exp(-acc))
        return carry

    lax.fori_loop(0, BPW // 16, p2_body, 0)

    pltpu.sync_copy(out_v, out_hbm.at[pl.ds(base, BPW)])


def kernel(user, item, age, income, user_table, item_table,
           age_table, income_table, fc_w, fc_b):
    w = jnp.concatenate([
        fc_w.reshape(-1).astype(jnp.float32),
        fc_b.reshape(-1).astype(jnp.float32),
        jnp.zeros((7,), jnp.float32),
    ])
    return _sc_recommender(
        user.astype(jnp.int32), item.astype(jnp.int32),
        age.astype(jnp.int32), income.astype(jnp.int32),
        user_table, item_table, age_table, income_table, w)
